# initial kernel scaffold (unmeasured)
import jax
import jax.numpy as jnp
from jax import lax
from jax.experimental import pallas as pl
from jax.experimental.pallas import tpu as pltpu


def kernel(
    x,
):
    def body(*refs):
        pass

    out_shape = jax.ShapeDtypeStruct(..., jnp.float32)
    return pl.pallas_call(body, out_shape=out_shape)(...)


# baseline (device time: 160572 ns/iter reference)
import jax
import jax.numpy as jnp
from jax import lax
from jax.experimental import pallas as pl
from jax.experimental.pallas import tpu as pltpu

N_Z = 4


def kernel(x):
    m, n = x.shape
    rows = m // N_Z

    def body(x_ref, out_ref, tmp_ref, send_sems, recv_sems):
        xi = lax.axis_index("x")
        yi = lax.axis_index("y")
        zi = lax.axis_index("z")
        right = (zi + 1) % N_Z
        left = (zi + N_Z - 1) % N_Z

        barrier_sem = pltpu.get_barrier_semaphore()
        for nbr in (left, right):
            pl.semaphore_signal(
                barrier_sem,
                inc=1,
                device_id=(xi, yi, nbr),
                device_id_type=pl.DeviceIdType.MESH,
            )
        pl.semaphore_wait(barrier_sem, 2)

        out_ref[...] = x_ref[...].astype(out_ref.dtype)

        def chunk(ref, idx):
            return ref.at[pl.ds(idx * rows, rows), :]

        for t in range(N_Z - 1):
            s_idx = (zi - t) % N_Z
            rdma = pltpu.make_async_remote_copy(
                src_ref=chunk(out_ref, s_idx),
                dst_ref=tmp_ref.at[t],
                send_sem=send_sems.at[t],
                recv_sem=recv_sems.at[t],
                device_id=(xi, yi, right),
                device_id_type=pl.DeviceIdType.MESH,
            )
            rdma.start()
            rdma.wait()
            r_start = ((zi - t - 1) % N_Z) * rows
            out_ref[pl.ds(r_start, rows), :] = (
                out_ref[pl.ds(r_start, rows), :] + tmp_ref[t]
            )

        for t in range(N_Z - 1):
            g_idx = (zi + 1 - t) % N_Z
            rdma = pltpu.make_async_remote_copy(
                src_ref=chunk(out_ref, g_idx),
                dst_ref=chunk(out_ref, g_idx),
                send_sem=send_sems.at[N_Z - 1 + t],
                recv_sem=recv_sems.at[N_Z - 1 + t],
                device_id=(xi, yi, right),
                device_id_type=pl.DeviceIdType.MESH,
            )
            rdma.start()
            rdma.wait()

    n_steps = 2 * (N_Z - 1)
    return pl.pallas_call(
        body,
        out_shape=jax.ShapeDtypeStruct((m, n), jnp.bfloat16),
        in_specs=[pl.BlockSpec(memory_space=pltpu.VMEM)],
        out_specs=pl.BlockSpec(memory_space=pltpu.VMEM),
        scratch_shapes=[
            pltpu.VMEM((N_Z - 1, rows, n), jnp.bfloat16),
            pltpu.SemaphoreType.DMA((n_steps,)),
            pltpu.SemaphoreType.DMA((n_steps,)),
        ],
        compiler_params=pltpu.CompilerParams(collective_id=0),
    )(x)


# device time: 94564 ns/iter; 1.6980x vs baseline; 1.6980x over previous
import jax
import jax.numpy as jnp
from jax import lax
from jax.experimental import pallas as pl
from jax.experimental.pallas import tpu as pltpu

N_Z = 4
N_P = 8

_MESH = pl.DeviceIdType.MESH


def _snake_coords(p):
    xx = p // 4
    yy = jnp.where(xx == 0, p, 7 - p)
    return xx, yy


def kernel(x):
    m, n = x.shape
    blk = m // N_P
    half = blk // 2
    zch = blk // N_Z

    def body(x_ref, out_ref, tmp_ref, a_send, a_recv, p_send, p_recv, q_send, q_recv):
        xi = lax.axis_index("x")
        yi = lax.axis_index("y")
        zi = lax.axis_index("z")
        zright = (zi + 1) % N_Z
        zleft = (zi + N_Z - 1) % N_Z

        p = jnp.where(xi == 0, yi, 7 - yi)
        nx, ny = _snake_coords((p + 1) % N_P)
        px, py = _snake_coords((p + N_P - 1) % N_P)

        barrier_sem = pltpu.get_barrier_semaphore()
        for dev in ((xi, yi, zleft), (xi, yi, zright), (nx, ny, zi), (px, py, zi)):
            pl.semaphore_signal(
                barrier_sem, inc=1, device_id=dev, device_id_type=_MESH
            )
        pl.semaphore_wait(barrier_sem, 4)

        base = p * blk
        out_ref[pl.ds(base, blk), :] = x_ref[pl.ds(base, blk), :].astype(
            out_ref.dtype
        )

        for t in range(N_Z - 1):
            s_start = base + ((zi - t) % N_Z) * zch
            rdma = pltpu.make_async_remote_copy(
                src_ref=out_ref.at[pl.ds(s_start, zch), :],
                dst_ref=tmp_ref.at[t],
                send_sem=a_send.at[t],
                recv_sem=a_recv.at[t],
                device_id=(xi, yi, zright),
                device_id_type=_MESH,
            )
            rdma.start()
            rdma.wait()
            r_start = base + ((zi - t - 1) % N_Z) * zch
            out_ref[pl.ds(r_start, zch), :] = (
                out_ref[pl.ds(r_start, zch), :] + tmp_ref[t]
            )
        for t in range(N_Z - 1):
            g_start = base + ((zi + 1 - t) % N_Z) * zch
            rdma = pltpu.make_async_remote_copy(
                src_ref=out_ref.at[pl.ds(g_start, zch), :],
                dst_ref=out_ref.at[pl.ds(g_start, zch), :],
                send_sem=a_send.at[N_Z - 1 + t],
                recv_sem=a_recv.at[N_Z - 1 + t],
                device_id=(xi, yi, zright),
                device_id_type=_MESH,
            )
            rdma.start()
            rdma.wait()

        for t in range(N_P - 1):
            fwd_start = ((p - t) % N_P) * blk
            rdma_f = pltpu.make_async_remote_copy(
                src_ref=out_ref.at[pl.ds(fwd_start, half), :],
                dst_ref=out_ref.at[pl.ds(fwd_start, half), :],
                send_sem=p_send.at[t],
                recv_sem=p_recv.at[t],
                device_id=(nx, ny, zi),
                device_id_type=_MESH,
            )
            bwd_start = ((p + t) % N_P) * blk + half
            rdma_b = pltpu.make_async_remote_copy(
                src_ref=out_ref.at[pl.ds(bwd_start, half), :],
                dst_ref=out_ref.at[pl.ds(bwd_start, half), :],
                send_sem=q_send.at[t],
                recv_sem=q_recv.at[t],
                device_id=(px, py, zi),
                device_id_type=_MESH,
            )
            rdma_f.start()
            rdma_b.start()
            rdma_f.wait()
            rdma_b.wait()

    return pl.pallas_call(
        body,
        out_shape=jax.ShapeDtypeStruct((m, n), jnp.bfloat16),
        in_specs=[pl.BlockSpec(memory_space=pltpu.VMEM)],
        out_specs=pl.BlockSpec(memory_space=pltpu.VMEM),
        scratch_shapes=[
            pltpu.VMEM((N_Z - 1, zch, n), jnp.bfloat16),
            pltpu.SemaphoreType.DMA((2 * (N_Z - 1),)),
            pltpu.SemaphoreType.DMA((2 * (N_Z - 1),)),
            pltpu.SemaphoreType.DMA((N_P - 1,)),
            pltpu.SemaphoreType.DMA((N_P - 1,)),
            pltpu.SemaphoreType.DMA((N_P - 1,)),
            pltpu.SemaphoreType.DMA((N_P - 1,)),
        ],
        compiler_params=pltpu.CompilerParams(collective_id=0),
    )(x)


# device time: 84797 ns/iter; 1.8936x vs baseline; 1.1152x over previous
import jax
import jax.numpy as jnp
from jax import lax
from jax.experimental import pallas as pl
from jax.experimental.pallas import tpu as pltpu

N_Z = 4
N_P = 8
N_B = N_P - 1

_MESH = pl.DeviceIdType.MESH


def _snake_coords(p):
    xx = p // 4
    yy = jnp.where(xx == 0, p, 7 - p)
    return xx, yy


def kernel(x):
    m, n = x.shape
    blk = m // N_P
    zch = blk // N_Z

    def body(x_ref, out_ref, tmp_ref, a_send, a_recv, b_send, b_recv):
        xi = lax.axis_index("x")
        yi = lax.axis_index("y")
        zi = lax.axis_index("z")
        zright = (zi + 1) % N_Z
        zleft = (zi + N_Z - 1) % N_Z

        p = jnp.where(xi == 0, yi, 7 - yi)
        nx, ny = _snake_coords((p + 1) % N_P)
        px, py = _snake_coords((p + N_P - 1) % N_P)

        barrier_sem = pltpu.get_barrier_semaphore()
        for dev in ((xi, yi, zleft), (xi, yi, zright), (nx, ny, zi), (px, py, zi)):
            pl.semaphore_signal(
                barrier_sem, inc=1, device_id=dev, device_id_type=_MESH
            )
        pl.semaphore_wait(barrier_sem, 4)

        base = p * blk
        out_ref[pl.ds(base, blk), :] = x_ref[pl.ds(base, blk), :].astype(
            out_ref.dtype
        )

        def rs_rdma(t):
            s_start = base + ((zi - t) % N_Z) * zch
            return pltpu.make_async_remote_copy(
                src_ref=out_ref.at[pl.ds(s_start, zch), :],
                dst_ref=tmp_ref.at[t],
                send_sem=a_send.at[t],
                recv_sem=a_recv.at[t],
                device_id=(xi, yi, zright),
                device_id_type=_MESH,
            )

        def ag_rdma(t):
            g_start = base + ((zi + 1 - t) % N_Z) * zch
            sl = pl.ds(g_start, zch)
            return pltpu.make_async_remote_copy(
                src_ref=out_ref.at[sl, :],
                dst_ref=out_ref.at[sl, :],
                send_sem=a_send.at[N_Z - 1 + t],
                recv_sem=a_recv.at[N_Z - 1 + t],
                device_id=(xi, yi, zright),
                device_id_type=_MESH,
            )

        def sub_rdma(c, t):
            if c < 2:
                blk_idx = (p - t) % N_P
                tgt = (nx, ny, zi)
            else:
                blk_idx = (p + t) % N_P
                tgt = (px, py, zi)
            sl = pl.ds(blk_idx * blk + c * zch, zch)
            return pltpu.make_async_remote_copy(
                src_ref=out_ref.at[sl, :],
                dst_ref=out_ref.at[sl, :],
                send_sem=b_send.at[c * N_B + t],
                recv_sem=b_recv.at[c * N_B + t],
                device_id=tgt,
                device_id_type=_MESH,
            )

        for t in range(N_Z - 1):
            r = rs_rdma(t)
            r.start()
            r.wait_recv()
            r_start = base + ((zi - t - 1) % N_Z) * zch
            out_ref[pl.ds(r_start, zch), :] = (
                out_ref[pl.ds(r_start, zch), :] + tmp_ref[t]
            )
        for t in range(N_Z - 1):
            ag_rdma(t).start()
            ag_rdma(t).wait_recv()

        for c in range(N_Z):
            sub_rdma(c, 0).start()
        for t in range(N_B - 1):
            for c in range(N_Z):
                sub_rdma(c, t).wait_recv()
                sub_rdma(c, t + 1).start()
        for c in range(N_Z):
            sub_rdma(c, N_B - 1).wait_recv()

        for t in range(2 * (N_Z - 1)):
            (rs_rdma(t) if t < N_Z - 1 else ag_rdma(t - (N_Z - 1))).wait_send()
        for c in range(N_Z):
            for t in range(N_B):
                sub_rdma(c, t).wait_send()

    return pl.pallas_call(
        body,
        out_shape=jax.ShapeDtypeStruct((m, n), jnp.bfloat16),
        in_specs=[pl.BlockSpec(memory_space=pltpu.VMEM)],
        out_specs=pl.BlockSpec(memory_space=pltpu.VMEM),
        scratch_shapes=[
            pltpu.VMEM((N_Z - 1, zch, n), jnp.bfloat16),
            pltpu.SemaphoreType.DMA((2 * (N_Z - 1),)),
            pltpu.SemaphoreType.DMA((2 * (N_Z - 1),)),
            pltpu.SemaphoreType.DMA((N_Z * N_B,)),
            pltpu.SemaphoreType.DMA((N_Z * N_B,)),
        ],
        compiler_params=pltpu.CompilerParams(collective_id=0),
    )(x)


# device time: 80665 ns/iter; 1.9906x vs baseline; 1.0512x over previous
import jax
import jax.numpy as jnp
from jax import lax
from jax.experimental import pallas as pl
from jax.experimental.pallas import tpu as pltpu

N_Z = 4
N_P = 8
N_B = N_P - 1

_MESH = pl.DeviceIdType.MESH


def _snake_coords(p):
    xx = p // 4
    yy = jnp.where(xx == 0, p, 7 - p)
    return xx, yy


def kernel(x):
    m, n = x.shape
    blk = m // N_P
    zch = blk // N_Z

    def body(x_ref, out_ref, tmp_ref, a_send, a_recv, b_send, b_recv):
        xi = lax.axis_index("x")
        yi = lax.axis_index("y")
        zi = lax.axis_index("z")
        zright = (zi + 1) % N_Z
        zleft = (zi + N_Z - 1) % N_Z

        p = jnp.where(xi == 0, yi, 7 - yi)
        nx, ny = _snake_coords((p + 1) % N_P)
        px, py = _snake_coords((p + N_P - 1) % N_P)

        barrier_sem = pltpu.get_barrier_semaphore()
        for dev in ((xi, yi, zleft), (xi, yi, zright), (nx, ny, zi), (px, py, zi)):
            pl.semaphore_signal(
                barrier_sem, inc=1, device_id=dev, device_id_type=_MESH
            )
        pl.semaphore_wait(barrier_sem, 4)

        base = p * blk
        out_ref[pl.ds(base, blk), :] = x_ref[pl.ds(base, blk), :].astype(
            out_ref.dtype
        )

        def rs_rdma(t):
            s_start = base + ((zi - t) % N_Z) * zch
            return pltpu.make_async_remote_copy(
                src_ref=out_ref.at[pl.ds(s_start, zch), :],
                dst_ref=tmp_ref.at[t],
                send_sem=a_send.at[t],
                recv_sem=a_recv.at[t],
                device_id=(xi, yi, zright),
                device_id_type=_MESH,
            )

        def ag_rdma(t):
            g_start = base + ((zi + 1 - t) % N_Z) * zch
            sl = pl.ds(g_start, zch)
            return pltpu.make_async_remote_copy(
                src_ref=out_ref.at[sl, :],
                dst_ref=out_ref.at[sl, :],
                send_sem=a_send.at[N_Z - 1 + t],
                recv_sem=a_recv.at[N_Z - 1 + t],
                device_id=(xi, yi, zright),
                device_id_type=_MESH,
            )

        def sub_rdma(k, t):
            c = (zi + 1 - k) % N_Z
            fwd = c < 2
            blk_idx = jnp.where(fwd, (p - t) % N_P, (p + t) % N_P)
            tgt = (jnp.where(fwd, nx, px), jnp.where(fwd, ny, py), zi)
            sl = pl.ds(blk_idx * blk + c * zch, zch)
            return pltpu.make_async_remote_copy(
                src_ref=out_ref.at[sl, :],
                dst_ref=out_ref.at[sl, :],
                send_sem=b_send.at[k * N_B + t],
                recv_sem=b_recv.at[k * N_B + t],
                device_id=tgt,
                device_id_type=_MESH,
            )

        for t in range(N_Z - 1):
            r = rs_rdma(t)
            r.start()
            r.wait_recv()
            r_start = base + ((zi - t - 1) % N_Z) * zch
            out_ref[pl.ds(r_start, zch), :] = (
                out_ref[pl.ds(r_start, zch), :] + tmp_ref[t]
            )
        sub_rdma(0, 0).start()
        for t in range(N_Z - 1):
            ag_rdma(t).start()
            ag_rdma(t).wait_recv()
            sub_rdma(t + 1, 0).start()

        for t in range(N_B - 1):
            for k in range(N_Z):
                sub_rdma(k, t).wait_recv()
                sub_rdma(k, t + 1).start()
        for k in range(N_Z):
            sub_rdma(k, N_B - 1).wait_recv()

        for t in range(2 * (N_Z - 1)):
            (rs_rdma(t) if t < N_Z - 1 else ag_rdma(t - (N_Z - 1))).wait_send()
        for k in range(N_Z):
            for t in range(N_B):
                sub_rdma(k, t).wait_send()

    return pl.pallas_call(
        body,
        out_shape=jax.ShapeDtypeStruct((m, n), jnp.bfloat16),
        in_specs=[pl.BlockSpec(memory_space=pltpu.VMEM)],
        out_specs=pl.BlockSpec(memory_space=pltpu.VMEM),
        scratch_shapes=[
            pltpu.VMEM((N_Z - 1, zch, n), jnp.bfloat16),
            pltpu.SemaphoreType.DMA((2 * (N_Z - 1),)),
            pltpu.SemaphoreType.DMA((2 * (N_Z - 1),)),
            pltpu.SemaphoreType.DMA((N_Z * N_B,)),
            pltpu.SemaphoreType.DMA((N_Z * N_B,)),
        ],
        compiler_params=pltpu.CompilerParams(collective_id=0),
    )(x)


# device time: 72639 ns/iter; 2.2105x vs baseline; 1.1105x over previous
import jax
import jax.numpy as jnp
from jax import lax
from jax.experimental import pallas as pl
from jax.experimental.pallas import tpu as pltpu

N_Z = 4
N_P = 8
N_B = N_P - 1

_MESH = pl.DeviceIdType.MESH


def _snake_coords(p):
    xx = p // 4
    yy = jnp.where(xx == 0, p, 7 - p)
    return xx, yy


def kernel(x):
    m, n = x.shape
    blk = m // N_P
    zch = blk // N_Z
    hch = zch // 2

    xi0 = lax.axis_index("x")
    yi0 = lax.axis_index("y")
    p0 = jnp.where(xi0 == 0, yi0, 7 - yi0)
    x_blk = lax.dynamic_slice(x, (p0 * blk, 0), (blk, n)).astype(jnp.bfloat16)

    def body(x_ref, out_ref, tmp_ref, a_send, a_recv, b_send, b_recv):
        xi = lax.axis_index("x")
        yi = lax.axis_index("y")
        zi = lax.axis_index("z")
        zright = (zi + 1) % N_Z
        zleft = (zi + N_Z - 1) % N_Z

        p = jnp.where(xi == 0, yi, 7 - yi)
        nx, ny = _snake_coords((p + 1) % N_P)
        px, py = _snake_coords((p + N_P - 1) % N_P)

        barrier_sem = pltpu.get_barrier_semaphore()
        for dev in ((xi, yi, zleft), (xi, yi, zright), (nx, ny, zi), (px, py, zi)):
            pl.semaphore_signal(
                barrier_sem, inc=1, device_id=dev, device_id_type=_MESH
            )
        pl.semaphore_wait(barrier_sem, 4)

        base = p * blk
        out_ref[pl.ds(base, blk), :] = x_ref[...]

        def rs_rdma(t, u):
            s_start = base + ((zi - t) % N_Z) * zch + u * hch
            return pltpu.make_async_remote_copy(
                src_ref=out_ref.at[pl.ds(s_start, hch), :],
                dst_ref=tmp_ref.at[t, u],
                send_sem=a_send.at[2 * t + u],
                recv_sem=a_recv.at[2 * t + u],
                device_id=(xi, yi, zright),
                device_id_type=_MESH,
            )

        def ag_rdma(t):
            g_start = base + ((zi + 1 - t) % N_Z) * zch
            sl = pl.ds(g_start, zch)
            return pltpu.make_async_remote_copy(
                src_ref=out_ref.at[sl, :],
                dst_ref=out_ref.at[sl, :],
                send_sem=a_send.at[2 * (N_Z - 1) + t],
                recv_sem=a_recv.at[2 * (N_Z - 1) + t],
                device_id=(xi, yi, zright),
                device_id_type=_MESH,
            )

        def sub_rdma(k, t):
            c = (zi + 1 - k) % N_Z
            fwd = c < 2
            blk_idx = jnp.where(fwd, (p - t) % N_P, (p + t) % N_P)
            tgt = (jnp.where(fwd, nx, px), jnp.where(fwd, ny, py), zi)
            sl = pl.ds(blk_idx * blk + c * zch, zch)
            return pltpu.make_async_remote_copy(
                src_ref=out_ref.at[sl, :],
                dst_ref=out_ref.at[sl, :],
                send_sem=b_send.at[k * N_B + t],
                recv_sem=b_recv.at[k * N_B + t],
                device_id=tgt,
                device_id_type=_MESH,
            )

        for u in range(2):
            rs_rdma(0, u).start()
        for t in range(N_Z - 1):
            r_base = base + ((zi - t - 1) % N_Z) * zch
            for u in range(2):
                rs_rdma(t, u).wait_recv()
                sl = pl.ds(r_base + u * hch, hch)
                out_ref[sl, :] = out_ref[sl, :] + tmp_ref[t, u]
                if t < N_Z - 2:
                    rs_rdma(t + 1, u).start()

        sub_rdma(0, 0).start()
        for t in range(N_Z - 1):
            ag_rdma(t).start()
            ag_rdma(t).wait_recv()
            sub_rdma(t + 1, 0).start()

        for t in range(N_B - 1):
            for k in range(N_Z):
                sub_rdma(k, t).wait_recv()
                sub_rdma(k, t + 1).start()
        for k in range(N_Z):
            sub_rdma(k, N_B - 1).wait_recv()

        for t in range(N_Z - 1):
            for u in range(2):
                rs_rdma(t, u).wait_send()
        for t in range(N_Z - 1):
            ag_rdma(t).wait_send()
        for k in range(N_Z):
            for t in range(N_B):
                sub_rdma(k, t).wait_send()

    return pl.pallas_call(
        body,
        out_shape=jax.ShapeDtypeStruct((m, n), jnp.bfloat16),
        in_specs=[pl.BlockSpec(memory_space=pltpu.VMEM)],
        out_specs=pl.BlockSpec(memory_space=pltpu.VMEM),
        scratch_shapes=[
            pltpu.VMEM((N_Z - 1, 2, hch, n), jnp.bfloat16),
            pltpu.SemaphoreType.DMA((3 * (N_Z - 1),)),
            pltpu.SemaphoreType.DMA((3 * (N_Z - 1),)),
            pltpu.SemaphoreType.DMA((N_Z * N_B,)),
            pltpu.SemaphoreType.DMA((N_Z * N_B,)),
        ],
        compiler_params=pltpu.CompilerParams(collective_id=0),
    )(x_blk)


# device time: 63624 ns/iter; 2.5238x vs baseline; 1.1417x over previous
import jax
import jax.numpy as jnp
from jax import lax
from jax.experimental import pallas as pl
from jax.experimental.pallas import tpu as pltpu

N_Z = 4
N_P = 8
N_OFF = 2
N_B = N_P - 1 - N_OFF

_MESH = pl.DeviceIdType.MESH


def _snake_coords(p):
    xx = p // 4
    yy = jnp.where(xx == 0, p, 7 - p)
    return xx, yy


def kernel(x):
    m, n = x.shape
    blk = m // N_P
    zch = blk // N_Z
    hch = zch // 2

    xi0 = lax.axis_index("x")
    yi0 = lax.axis_index("y")
    p0 = jnp.where(xi0 == 0, yi0, 7 - yi0)
    x_blk = lax.dynamic_slice(x, (p0 * blk, 0), (blk, n)).astype(jnp.bfloat16)

    def body(
        x_ref, out_ref, tmp_ref, a_send, a_recv, b_send, b_recv, z_send, z_recv
    ):
        xi = lax.axis_index("x")
        yi = lax.axis_index("y")
        zi = lax.axis_index("z")
        zright = (zi + 1) % N_Z
        zleft = (zi + N_Z - 1) % N_Z

        p = jnp.where(xi == 0, yi, 7 - yi)
        nx, ny = _snake_coords((p + 1) % N_P)
        px, py = _snake_coords((p + N_P - 1) % N_P)

        barrier_sem = pltpu.get_barrier_semaphore()
        for dev in ((xi, yi, zleft), (xi, yi, zright), (nx, ny, zi), (px, py, zi)):
            pl.semaphore_signal(
                barrier_sem, inc=1, device_id=dev, device_id_type=_MESH
            )
        pl.semaphore_wait(barrier_sem, 4)

        base = p * blk
        out_ref[pl.ds(base, blk), :] = x_ref[...]

        def rs_rdma(t, u):
            s_start = base + ((zi - t) % N_Z) * zch + u * hch
            return pltpu.make_async_remote_copy(
                src_ref=out_ref.at[pl.ds(s_start, hch), :],
                dst_ref=tmp_ref.at[t, u],
                send_sem=a_send.at[2 * t + u],
                recv_sem=a_recv.at[2 * t + u],
                device_id=(xi, yi, zright),
                device_id_type=_MESH,
            )

        def ag_rdma(t):
            g_start = base + ((zi + 1 - t) % N_Z) * zch
            sl = pl.ds(g_start, zch)
            return pltpu.make_async_remote_copy(
                src_ref=out_ref.at[sl, :],
                dst_ref=out_ref.at[sl, :],
                send_sem=a_send.at[2 * (N_Z - 1) + t],
                recv_sem=a_recv.at[2 * (N_Z - 1) + t],
                device_id=(xi, yi, zright),
                device_id_type=_MESH,
            )

        def sub_fwd(k):
            c = (zi + 1 - k) % N_Z
            return jnp.logical_xor(c < 2, zi % 2 == 1)

        def sub_rdma(k, t):
            c = (zi + 1 - k) % N_Z
            fwd = sub_fwd(k)
            blk_idx = jnp.where(fwd, (p - t) % N_P, (p + t) % N_P)
            tgt = (jnp.where(fwd, nx, px), jnp.where(fwd, ny, py), zi)
            sl = pl.ds(blk_idx * blk + c * zch, zch)
            return pltpu.make_async_remote_copy(
                src_ref=out_ref.at[sl, :],
                dst_ref=out_ref.at[sl, :],
                send_sem=b_send.at[k * N_B + t],
                recv_sem=b_recv.at[k * N_B + t],
                device_id=tgt,
                device_id_type=_MESH,
            )

        def _chunk_fwd(c):
            return jnp.logical_xor(c < 2, zi % 2 == 1)

        def z_rdma(c, j):
            fwd = _chunk_fwd(c)
            blk_idx = jnp.where(fwd, (p - (j + 1)) % N_P, (p + (j + 1)) % N_P)
            sl = pl.ds(blk_idx * blk + c * zch, zch)
            return pltpu.make_async_remote_copy(
                src_ref=out_ref.at[sl, :],
                dst_ref=out_ref.at[sl, :],
                send_sem=z_send.at[c * N_OFF + j],
                recv_sem=z_recv.at[c * N_OFF + j],
                device_id=(xi, yi, zi ^ 1),
                device_id_type=_MESH,
            )

        def z_wait_rdma(c, j):
            fwd = _chunk_fwd(c)
            blk_idx = jnp.where(fwd, (p + (j + 1)) % N_P, (p - (j + 1)) % N_P)
            sl = pl.ds(blk_idx * blk + c * zch, zch)
            return pltpu.make_async_remote_copy(
                src_ref=out_ref.at[sl, :],
                dst_ref=out_ref.at[sl, :],
                send_sem=z_send.at[c * N_OFF + j],
                recv_sem=z_recv.at[c * N_OFF + j],
                device_id=(xi, yi, zi ^ 1),
                device_id_type=_MESH,
            )

        for u in range(2):
            rs_rdma(0, u).start()
        for t in range(N_Z - 1):
            r_base = base + ((zi - t - 1) % N_Z) * zch
            for u in range(2):
                rs_rdma(t, u).wait_recv()
                sl = pl.ds(r_base + u * hch, hch)
                out_ref[sl, :] = out_ref[sl, :] + tmp_ref[t, u]
                if t < N_Z - 2:
                    rs_rdma(t + 1, u).start()

        sub_rdma(0, 0).start()
        for t in range(N_Z - 1):
            ag_rdma(t).start()
            ag_rdma(t).wait_recv()
            sub_rdma(t + 1, 0).start()

        for t in range(N_B - 1):
            for k in range(N_Z):
                sub_rdma(k, t).wait_recv()
                sub_rdma(k, t + 1).start()
            if t < N_OFF:
                for c in range(N_Z):
                    z_rdma(c, t).start()
        for k in range(N_Z):
            sub_rdma(k, N_B - 1).wait_recv()
        for c in range(N_Z):
            for j in range(N_OFF):
                z_wait_rdma(c, j).wait_recv()

        for t in range(N_Z - 1):
            for u in range(2):
                rs_rdma(t, u).wait_send()
        for t in range(N_Z - 1):
            ag_rdma(t).wait_send()
        for k in range(N_Z):
            for t in range(N_B):
                sub_rdma(k, t).wait_send()
        for c in range(N_Z):
            for j in range(N_OFF):
                z_rdma(c, j).wait_send()

    return pl.pallas_call(
        body,
        out_shape=jax.ShapeDtypeStruct((m, n), jnp.bfloat16),
        in_specs=[pl.BlockSpec(memory_space=pltpu.VMEM)],
        out_specs=pl.BlockSpec(memory_space=pltpu.VMEM),
        scratch_shapes=[
            pltpu.VMEM((N_Z - 1, 2, hch, n), jnp.bfloat16),
            pltpu.SemaphoreType.DMA((3 * (N_Z - 1),)),
            pltpu.SemaphoreType.DMA((3 * (N_Z - 1),)),
            pltpu.SemaphoreType.DMA((N_Z * N_B,)),
            pltpu.SemaphoreType.DMA((N_Z * N_B,)),
            pltpu.SemaphoreType.DMA((N_Z * N_OFF,)),
            pltpu.SemaphoreType.DMA((N_Z * N_OFF,)),
        ],
        compiler_params=pltpu.CompilerParams(collective_id=0),
    )(x_blk)


# device time: 60888 ns/iter; 2.6372x vs baseline; 1.0449x over previous
import jax
import jax.numpy as jnp
from jax import lax
from jax.experimental import pallas as pl
from jax.experimental.pallas import tpu as pltpu

N_Z = 4
N_P = 8
N_OFF = 2
N_B = N_P - 1 - N_OFF

_MESH = pl.DeviceIdType.MESH


def _snake_coords(p):
    xx = p // 4
    yy = jnp.where(xx == 0, p, 7 - p)
    return xx, yy


def kernel(x):
    m, n = x.shape
    blk = m // N_P
    zch = blk // N_Z
    hch = zch // 2

    xi0 = lax.axis_index("x")
    yi0 = lax.axis_index("y")
    p0 = jnp.where(xi0 == 0, yi0, 7 - yi0)
    x_blk = lax.dynamic_slice(x, (p0 * blk, 0), (blk, n)).astype(jnp.bfloat16)

    def body(
        x_ref, out_ref, tmp_ref, a_send, a_recv, b_send, b_recv, z_send, z_recv
    ):
        xi = lax.axis_index("x")
        yi = lax.axis_index("y")
        zi = lax.axis_index("z")
        zright = (zi + 1) % N_Z
        zleft = (zi + N_Z - 1) % N_Z

        p = jnp.where(xi == 0, yi, 7 - yi)
        nx, ny = _snake_coords((p + 1) % N_P)
        px, py = _snake_coords((p + N_P - 1) % N_P)

        barrier_sem = pltpu.get_barrier_semaphore()
        for dev in ((xi, yi, zleft), (xi, yi, zright), (nx, ny, zi), (px, py, zi)):
            pl.semaphore_signal(
                barrier_sem, inc=1, device_id=dev, device_id_type=_MESH
            )
        pl.semaphore_wait(barrier_sem, 4)

        base = p * blk
        out_ref[pl.ds(base, blk), :] = x_ref[...]

        def rs_rdma(t, u):
            s_start = base + ((zi - t) % N_Z) * zch + u * hch
            return pltpu.make_async_remote_copy(
                src_ref=out_ref.at[pl.ds(s_start, hch), :],
                dst_ref=tmp_ref.at[t, u],
                send_sem=a_send.at[2 * t + u],
                recv_sem=a_recv.at[2 * t + u],
                device_id=(xi, yi, zright),
                device_id_type=_MESH,
            )

        def ag_rdma(t, u):
            g_start = base + ((zi + 1 - t) % N_Z) * zch + u * hch
            sl = pl.ds(g_start, hch)
            return pltpu.make_async_remote_copy(
                src_ref=out_ref.at[sl, :],
                dst_ref=out_ref.at[sl, :],
                send_sem=a_send.at[2 * (N_Z - 1) + 2 * t + u],
                recv_sem=a_recv.at[2 * (N_Z - 1) + 2 * t + u],
                device_id=(xi, yi, zright),
                device_id_type=_MESH,
            )

        def sub_fwd(k):
            c = (zi + 1 - k) % N_Z
            return jnp.logical_xor(c < 2, zi % 2 == 1)

        def sub_rdma(k, t):
            c = (zi + 1 - k) % N_Z
            fwd = sub_fwd(k)
            blk_idx = jnp.where(fwd, (p - t) % N_P, (p + t) % N_P)
            tgt = (jnp.where(fwd, nx, px), jnp.where(fwd, ny, py), zi)
            sl = pl.ds(blk_idx * blk + c * zch, zch)
            return pltpu.make_async_remote_copy(
                src_ref=out_ref.at[sl, :],
                dst_ref=out_ref.at[sl, :],
                send_sem=b_send.at[k * N_B + t],
                recv_sem=b_recv.at[k * N_B + t],
                device_id=tgt,
                device_id_type=_MESH,
            )

        def _chunk_fwd(c):
            return jnp.logical_xor(c < 2, zi % 2 == 1)

        def z_rdma(c, j):
            fwd = _chunk_fwd(c)
            blk_idx = jnp.where(fwd, (p - (j + 1)) % N_P, (p + (j + 1)) % N_P)
            sl = pl.ds(blk_idx * blk + c * zch, zch)
            return pltpu.make_async_remote_copy(
                src_ref=out_ref.at[sl, :],
                dst_ref=out_ref.at[sl, :],
                send_sem=z_send.at[c * N_OFF + j],
                recv_sem=z_recv.at[c * N_OFF + j],
                device_id=(xi, yi, zi ^ 1),
                device_id_type=_MESH,
            )

        def z_wait_rdma(c, j):
            fwd = _chunk_fwd(c)
            blk_idx = jnp.where(fwd, (p + (j + 1)) % N_P, (p - (j + 1)) % N_P)
            sl = pl.ds(blk_idx * blk + c * zch, zch)
            return pltpu.make_async_remote_copy(
                src_ref=out_ref.at[sl, :],
                dst_ref=out_ref.at[sl, :],
                send_sem=z_send.at[c * N_OFF + j],
                recv_sem=z_recv.at[c * N_OFF + j],
                device_id=(xi, yi, zi ^ 1),
                device_id_type=_MESH,
            )

        for u in range(2):
            rs_rdma(0, u).start()
        for t in range(N_Z - 1):
            r_base = base + ((zi - t - 1) % N_Z) * zch
            for u in range(2):
                rs_rdma(t, u).wait_recv()
                sl = pl.ds(r_base + u * hch, hch)
                out_ref[sl, :] = out_ref[sl, :] + tmp_ref[t, u]
                if t < N_Z - 2:
                    rs_rdma(t + 1, u).start()

        sub_rdma(0, 0).start()
        for u in range(2):
            ag_rdma(0, u).start()
        for t in range(N_Z - 1):
            for u in range(2):
                ag_rdma(t, u).wait_recv()
                if t < N_Z - 2:
                    ag_rdma(t + 1, u).start()
            sub_rdma(t + 1, 0).start()

        for t in range(N_B - 1):
            for k in range(N_Z):
                sub_rdma(k, t).wait_recv()
                sub_rdma(k, t + 1).start()
            if t < N_OFF:
                for c in range(N_Z):
                    z_rdma(c, t).start()
        for k in range(N_Z):
            sub_rdma(k, N_B - 1).wait_recv()
        for c in range(N_Z):
            for j in range(N_OFF):
                z_wait_rdma(c, j).wait_recv()

        for t in range(N_Z - 1):
            for u in range(2):
                rs_rdma(t, u).wait_send()
        for t in range(N_Z - 1):
            for u in range(2):
                ag_rdma(t, u).wait_send()
        for k in range(N_Z):
            for t in range(N_B):
                sub_rdma(k, t).wait_send()
        for c in range(N_Z):
            for j in range(N_OFF):
                z_rdma(c, j).wait_send()

    return pl.pallas_call(
        body,
        out_shape=jax.ShapeDtypeStruct((m, n), jnp.bfloat16),
        in_specs=[pl.BlockSpec(memory_space=pltpu.VMEM)],
        out_specs=pl.BlockSpec(memory_space=pltpu.VMEM),
        scratch_shapes=[
            pltpu.VMEM((N_Z - 1, 2, hch, n), jnp.bfloat16),
            pltpu.SemaphoreType.DMA((4 * (N_Z - 1),)),
            pltpu.SemaphoreType.DMA((4 * (N_Z - 1),)),
            pltpu.SemaphoreType.DMA((N_Z * N_B,)),
            pltpu.SemaphoreType.DMA((N_Z * N_B,)),
            pltpu.SemaphoreType.DMA((N_Z * N_OFF,)),
            pltpu.SemaphoreType.DMA((N_Z * N_OFF,)),
        ],
        compiler_params=pltpu.CompilerParams(collective_id=0),
    )(x_blk)
